# baseline (speedup 1.0000x reference)
import functools
import jax
import jax.numpy as jnp
from jax import lax
from jax.experimental import pallas as pl
from jax.experimental.pallas import tpu as pltpu
from jax.experimental.pallas import tpu_sc as plsc

N_NODES = 10000
F_IN = 128
HID = 128
NCLASS = 16

NC = 2
NS = 16
NW = NC * NS
CHUNK = 128
N_PAD = 10112
ROWS_PER_TILE = N_PAD // NS
WSCALE = 1.0 / 65535.0


def _mm_body(x_ref, w_ref, o_ref):
    o_ref[...] = jnp.dot(x_ref[...], w_ref[...],
                         preferred_element_type=jnp.float32)


def _dense_matmul(x, w):
    return pl.pallas_call(
        _mm_body,
        out_shape=jax.ShapeDtypeStruct((x.shape[0], w.shape[1]), jnp.float32),
    )(x, w)


def _spmm_body(h_hbm, spk_hbm, wpk_hbm, acc_out, c_out,
               spk_v, wpk_v, srcb, dstb, wbuf, zbuf, rows0, rows1,
               acc_sh, c_sh, g0, g1, s0, s1, csem):
    cid = lax.axis_index("c")
    sid = lax.axis_index("s")
    wid = sid * NC + cid
    n_chunks = wpk_hbm.shape[1]
    rows = (rows0, rows1)
    gsem = (g0, g1)
    ssem = (s0, s1)

    def zero_rows(buf):
        def zero_row(r, _):
            for f in range(8):
                buf[r, pl.ds(f * 16, 16)] = jnp.zeros((16,), jnp.float32)
            return _
        lax.fori_loop(0, CHUNK, zero_row, None)
    zero_rows(rows0)
    zero_rows(rows1)
    for k in range(2):
        for f in range(8):
            wbuf[k, pl.ds(f * 16, 16)] = jnp.zeros((16,), jnp.float32)
    for f in range(8):
        zbuf[pl.ds(f * 16, 16)] = jnp.zeros((16,), jnp.float32)

    base = sid * ROWS_PER_TILE
    for t in range(4):
        pltpu.sync_copy(rows0, acc_sh.at[pl.ds(base + t * CHUNK, CHUNK)])
        pltpu.sync_copy(zbuf, c_sh.at[pl.ds(base + t * CHUNK, CHUNK)])
    pltpu.sync_copy(rows0.at[pl.ds(0, 120)],
                    acc_sh.at[pl.ds(base + 4 * CHUNK, 120)])
    pltpu.sync_copy(zbuf.at[pl.ds(0, 120)],
                    c_sh.at[pl.ds(base + 4 * CHUNK, 120)])
    plsc.subcore_barrier()

    pltpu.sync_copy(spk_hbm.at[wid], spk_v)
    pltpu.sync_copy(wpk_hbm.at[wid], wpk_v)

    mask14 = jnp.full((16,), 0x3FFF, jnp.int32)
    mask16 = jnp.full((16,), 0xFFFF, jnp.int32)

    def unpack_sd(j, q):
        def blk(k, _):
            v = spk_v[j, pl.ds(k * 16, 16)]
            srcb[q, pl.ds(k * 16, 16)] = v & mask14
            dstb[q, pl.ds(k * 16, 16)] = (
                lax.shift_right_logical(v, 14) & mask14)
            return _
        lax.fori_loop(0, CHUNK // 16, blk, None)

    def gather(q, buf, sem):
        return pltpu.make_async_copy(h_hbm.at[srcb.at[q]], buf, sem)

    def scatter(q, buf, sem):
        return pltpu.make_async_copy(buf, acc_sh.at[dstb.at[q]], sem)

    def cscat(q):
        return pltpu.make_async_copy(wbuf.at[q], c_sh.at[srcb.at[q]], csem)

    def cscat_dummy():
        return pltpu.make_async_copy(zbuf, c_sh.at[srcb.at[0]], csem)

    def scale(buf, x, p):
        def blk(b2, _):
            wpk = wpk_v[x, pl.ds(b2 * 16, 16)]
            wlo = (wpk & mask16).astype(jnp.float32) * WSCALE
            whi = lax.shift_right_logical(wpk, 16).astype(jnp.float32) * WSCALE
            wbuf[p, pl.ds(b2 * 32, 16)] = wlo
            wbuf[p, pl.ds(b2 * 32 + 16, 16)] = whi
            for half, wv in ((0, wlo), (1, whi)):
                for l in range(16):
                    i = b2 * 32 + half * 16 + l
                    wb = jnp.full((16,), wv[l], jnp.float32)
                    for f in range(8):
                        sl = pl.ds(f * 16, 16)
                        buf[i, sl] = buf[i, sl] * wb
            return _
        lax.fori_loop(0, CHUNK // 32, blk, None)

    unpack_sd(0, 0)
    gather(0, rows0, g0).start()
    scatter(0, rows1, s1).start(add=True)
    cscat_dummy().start(add=True)

    def step(x, p):
        q = 1 - p
        scatter(q, rows[q], ssem[q]).wait()
        cscat(q).wait()
        unpack_sd(x + 1, q)
        gather(q, rows[q], gsem[q]).start()
        gather(p, rows[p], gsem[p]).wait()
        scale(rows[p], x, p)
        scatter(p, rows[p], ssem[p]).start(add=True)
        cscat(p).start(add=True)

    def pipe(x2, _):
        step(x2 * 2, 0)
        step(x2 * 2 + 1, 1)
        return _

    lax.fori_loop(0, n_chunks // 2, pipe, None)
    gather(0, rows[0], gsem[0]).wait()
    scatter(1, rows[1], ssem[1]).wait()
    cscat(1).wait()
    plsc.subcore_barrier()

    pltpu.sync_copy(acc_sh.at[pl.ds(base, ROWS_PER_TILE)],
                    acc_out.at[cid, pl.ds(base, ROWS_PER_TILE)])
    pltpu.sync_copy(c_sh.at[pl.ds(base, ROWS_PER_TILE)],
                    c_out.at[cid, pl.ds(base, ROWS_PER_TILE)])


def _spmm(h, spk3, wpk3):
    n_chunks = wpk3.shape[1]
    kern = functools.partial(
        pl.kernel,
        out_type=(
            jax.ShapeDtypeStruct((NC, N_PAD, HID), jnp.float32),
            jax.ShapeDtypeStruct((NC, N_PAD), jnp.float32),
        ),
        mesh=plsc.VectorSubcoreMesh(core_axis_name="c", subcore_axis_name="s"),
        compiler_params=pltpu.CompilerParams(use_tc_tiling_on_sc=False),
        scratch_types=[
            pltpu.VMEM((n_chunks + 1, CHUNK), jnp.int32),
            pltpu.VMEM((n_chunks, CHUNK // 2), jnp.int32),
            pltpu.VMEM((2, CHUNK), jnp.int32),
            pltpu.VMEM((2, CHUNK), jnp.int32),
            pltpu.VMEM((2, CHUNK), jnp.float32),
            pltpu.VMEM((CHUNK,), jnp.float32),
            pltpu.VMEM((CHUNK, HID), jnp.float32),
            pltpu.VMEM((CHUNK, HID), jnp.float32),
            pltpu.VMEM_SHARED((N_PAD, HID), jnp.float32),
            pltpu.VMEM_SHARED((N_PAD,), jnp.float32),
            pltpu.SemaphoreType.DMA,
            pltpu.SemaphoreType.DMA,
            pltpu.SemaphoreType.DMA,
            pltpu.SemaphoreType.DMA,
            pltpu.SemaphoreType.DMA,
        ],
    )(_spmm_body)
    return kern(h, spk3, wpk3)


def _reduce_body(a0_ref, a1_ref, c0_ref, c1_ref, w2_ref, o_ref):
    i = pl.program_id(0)
    h1 = jnp.maximum(a0_ref[...] + a1_ref[...], 0.0)
    s = jnp.sum(h1 * (c0_ref[...] + c1_ref[...]), axis=0)[None, :]
    val = jnp.dot(s, w2_ref[...],
                  preferred_element_type=jnp.float32) * (1.0 / N_NODES)

    @pl.when(i == 0)
    def _():
        o_ref[...] = val

    @pl.when(i > 0)
    def _():
        o_ref[...] = o_ref[...] + val


def _reduce(acc, c, w2):
    blk = N_PAD // 8
    return pl.pallas_call(
        _reduce_body,
        grid=(8,),
        in_specs=[
            pl.BlockSpec((blk, HID), lambda i: (i, 0)),
            pl.BlockSpec((blk, HID), lambda i: (i, 0)),
            pl.BlockSpec((blk, 1), lambda i: (i, 0)),
            pl.BlockSpec((blk, 1), lambda i: (i, 0)),
            pl.BlockSpec((HID, NCLASS), lambda i: (0, 0)),
        ],
        out_specs=pl.BlockSpec((1, NCLASS), lambda i: (0, 0)),
        out_shape=jax.ShapeDtypeStruct((1, NCLASS), jnp.float32),
    )(acc[0], acc[1], c[0].reshape(N_PAD, 1), c[1].reshape(N_PAD, 1), w2)


def kernel(x, edge_index, edge_weight, W1, W2):
    e = edge_weight.shape[0]
    per_tile = -(-e // (NW * 2 * CHUNK)) * 2 * CHUNK
    e_pad = per_tile * NW
    n_chunks = per_tile // CHUNK

    src = jnp.asarray(edge_index[0], jnp.int32)
    dst = jnp.asarray(edge_index[1], jnp.int32)
    w = jnp.asarray(edge_weight, jnp.float32)
    pad = e_pad - e
    srcp = jnp.pad(src, (0, pad))
    dstp = jnp.pad(dst, (0, pad))
    spk3 = jnp.pad(
        (srcp | (dstp << 14)).reshape(NW, n_chunks, CHUNK),
        ((0, 0), (0, 1), (0, 0)))
    wq = jnp.round(jnp.pad(w, (0, pad)) * 65535.0).astype(jnp.int32)
    wr = wq.reshape(NW, n_chunks, CHUNK // 32, 2, 16)
    wpk3 = (wr[:, :, :, 0, :] | (wr[:, :, :, 1, :] << 16)).reshape(
        NW, n_chunks, CHUNK // 2)

    h = _dense_matmul(x, W1)
    acc, c = _spmm(h, spk3, wpk3)
    return _reduce(acc, c, W2)


# speedup vs baseline: 1.6887x; 1.6887x over previous
import functools
import jax
import jax.numpy as jnp
from jax import lax
from jax.experimental import pallas as pl
from jax.experimental.pallas import tpu as pltpu
from jax.experimental.pallas import tpu_sc as plsc

N_NODES = 10000
F_IN = 128
HID = 128
NCLASS = 16

NC = 2
NS = 16
NW = NC * NS
CHUNK = 128
N_PAD = 10240
ROWS_PER_TILE = N_PAD // NS


def _mm_body(x_ref, w_ref, o_ref):
    o_ref[...] = jnp.dot(x_ref[...], w_ref[...],
                         preferred_element_type=jnp.float32)


def _dense_matmul(x, w):
    return pl.pallas_call(
        _mm_body,
        out_shape=jax.ShapeDtypeStruct((x.shape[0], w.shape[1]), jnp.float32),
    )(x, w)


def _spmm_body(h_hbm, src_hbm, dst_hbm, w_hbm, acc_out, c_out,
               src_v, dst_v, w_v, rows, acc_sh, c_sh, csem):
    cid = lax.axis_index("c")
    sid = lax.axis_index("s")
    wid = sid * NC + cid
    n_chunks = src_v.shape[0]

    def zero_row(r, _):
        for f in range(8):
            rows[r, pl.ds(f * 16, 16)] = jnp.zeros((16,), jnp.float32)
        return _
    lax.fori_loop(0, CHUNK, zero_row, None)
    for t in range(ROWS_PER_TILE // CHUNK):
        off = sid * ROWS_PER_TILE + t * CHUNK
        pltpu.sync_copy(rows, acc_sh.at[pl.ds(off, CHUNK)])
        pltpu.sync_copy(rows.at[0], c_sh.at[pl.ds(off, CHUNK)])
    plsc.subcore_barrier()

    pltpu.sync_copy(src_hbm.at[wid], src_v)
    pltpu.sync_copy(dst_hbm.at[wid], dst_v)
    pltpu.sync_copy(w_hbm.at[wid], w_v)

    def cscat(j):
        return pltpu.make_async_copy(w_v.at[j], c_sh.at[src_v.at[j]], csem)

    def process(j):
        pltpu.sync_copy(h_hbm.at[src_v.at[j]], rows)

        def scale_block(b, __):
            wvec = w_v[j, pl.ds(b * 16, 16)]
            for l in range(16):
                i = b * 16 + l
                wb = jnp.full((16,), wvec[l], jnp.float32)
                for f in range(8):
                    sl = pl.ds(f * 16, 16)
                    rows[i, sl] = rows[i, sl] * wb
            return __
        lax.fori_loop(0, CHUNK // 16, scale_block, None)

        pltpu.sync_copy(rows, acc_sh.at[dst_v.at[j]], add=True)

    process(0)
    cscat(0).start(add=True)

    def edge_chunk(j, _):
        process(j)
        cscat(0).wait()
        cscat(j).start(add=True)
        return _

    lax.fori_loop(1, n_chunks, edge_chunk, None)
    cscat(0).wait()
    plsc.subcore_barrier()

    off = sid * ROWS_PER_TILE
    pltpu.sync_copy(acc_sh.at[pl.ds(off, ROWS_PER_TILE)],
                    acc_out.at[cid, pl.ds(off, ROWS_PER_TILE)])
    pltpu.sync_copy(c_sh.at[pl.ds(off, ROWS_PER_TILE)],
                    c_out.at[cid, pl.ds(off, ROWS_PER_TILE)])


def _spmm(h, src3, dst3, w3):
    n_chunks = src3.shape[1]
    f = h.shape[1]
    kern = functools.partial(
        pl.kernel,
        out_type=(
            jax.ShapeDtypeStruct((NC, N_PAD, f), jnp.float32),
            jax.ShapeDtypeStruct((NC, N_PAD), jnp.float32),
        ),
        mesh=plsc.VectorSubcoreMesh(core_axis_name="c", subcore_axis_name="s"),
        scratch_types=[
            pltpu.VMEM((n_chunks, CHUNK), jnp.int32),
            pltpu.VMEM((n_chunks, CHUNK), jnp.int32),
            pltpu.VMEM((n_chunks, CHUNK), jnp.float32),
            pltpu.VMEM((CHUNK, f), jnp.float32),
            pltpu.VMEM_SHARED((N_PAD, f), jnp.float32),
            pltpu.VMEM_SHARED((N_PAD,), jnp.float32),
            pltpu.SemaphoreType.DMA,
        ],
    )(_spmm_body)
    return kern(h, src3, dst3, w3)


def _reduce_body(a0_ref, a1_ref, c0_ref, c1_ref, w2_ref, o_ref):
    i = pl.program_id(0)
    h1 = jnp.maximum(a0_ref[...] + a1_ref[...], 0.0)
    s = jnp.sum(h1 * (c0_ref[...] + c1_ref[...]), axis=0)[None, :]
    val = jnp.dot(s, w2_ref[...],
                  preferred_element_type=jnp.float32) * (1.0 / N_NODES)

    @pl.when(i == 0)
    def _():
        o_ref[...] = val

    @pl.when(i > 0)
    def _():
        o_ref[...] = o_ref[...] + val


def _reduce(acc, c, w2):
    blk = 1024
    grid = N_PAD // blk
    return pl.pallas_call(
        _reduce_body,
        grid=(grid,),
        in_specs=[
            pl.BlockSpec((blk, HID), lambda i: (i, 0)),
            pl.BlockSpec((blk, HID), lambda i: (i, 0)),
            pl.BlockSpec((blk, 1), lambda i: (i, 0)),
            pl.BlockSpec((blk, 1), lambda i: (i, 0)),
            pl.BlockSpec((HID, NCLASS), lambda i: (0, 0)),
        ],
        out_specs=pl.BlockSpec((1, NCLASS), lambda i: (0, 0)),
        out_shape=jax.ShapeDtypeStruct((1, NCLASS), jnp.float32),
    )(acc[0], acc[1], c[0].reshape(N_PAD, 1), c[1].reshape(N_PAD, 1), w2)


def kernel(x, edge_index, edge_weight, W1, W2):
    e = edge_weight.shape[0]
    per_tile = -(-e // (NW * CHUNK)) * CHUNK
    e_pad = per_tile * NW
    n_chunks = per_tile // CHUNK

    src = jnp.asarray(edge_index[0], jnp.int32)
    dst = jnp.asarray(edge_index[1], jnp.int32)
    w = jnp.asarray(edge_weight, jnp.float32)
    pad = e_pad - e
    src3 = jnp.pad(src, (0, pad)).reshape(NW, n_chunks, CHUNK)
    dst3 = jnp.pad(dst, (0, pad)).reshape(NW, n_chunks, CHUNK)
    w3 = jnp.pad(w, (0, pad)).reshape(NW, n_chunks, CHUNK)

    h = _dense_matmul(x, W1)
    acc, c = _spmm(h, src3, dst3, w3)
    return _reduce(acc, c, W2)
